# Initial kernel scaffold; baseline (speedup 1.0000x reference)
import jax
import jax.numpy as jnp
from jax.experimental import pallas as pl


def kernel(x, edge_index, edge_type, W1, root1, b1, W2, root2, b2):
    raise NotImplementedError("write your pallas kernel here")


# speedup vs baseline: 7.8729x; 7.8729x over previous
import functools

import jax
import jax.numpy as jnp
from jax import lax
from jax.experimental import pallas as pl
from jax.experimental.pallas import tpu as pltpu
from jax.experimental.pallas import tpu_sc as plsc

NC, NS, L = 2, 16, 16


def _count_kernel(R, EP, CNT_ROWS):
    per_worker = EP // (NC * NS)
    CH = 2560
    n_chunks = per_worker // CH
    zrows = CNT_ROWS // NS
    mesh = plsc.VectorSubcoreMesh(
        core_axis_name="c", subcore_axis_name="s",
        num_cores=NC, num_subcores=NS)

    @functools.partial(
        pl.kernel,
        out_type=jax.ShapeDtypeStruct((NC, CNT_ROWS, 128), jnp.int32),
        mesh=mesh,
        compiler_params=pltpu.CompilerParams(needs_layout_passes=False),
        scratch_types=[
            pltpu.VMEM((CNT_ROWS, 128), jnp.int32),
            pltpu.VMEM((CH,), jnp.int32),
            pltpu.VMEM((CH,), jnp.int32),
            pltpu.VMEM((CNT_ROWS // 128, 128), jnp.int32),
            pltpu.VMEM_SHARED((CNT_ROWS, 128), jnp.int32),
        ],
    )
    def body(dst_hbm, et_hbm, out_hbm, cnt_v, dst_v, et_v, ridx_v, cnt_sh):
        c = lax.axis_index("c")
        s = lax.axis_index("s")
        wid = c * NS + s
        iota = lax.iota(jnp.int32, L)
        zero16 = jnp.zeros((L,), jnp.int32)

        def zrow(i, _):
            for k in range(8):
                cnt_v[i, pl.ds(k * L, L)] = zero16
            return 0
        lax.fori_loop(0, CNT_ROWS, zrow, 0)
        for i in range(CNT_ROWS // 128):
            for k in range(8):
                ridx_v[i, pl.ds(k * L, L)] = iota + (i * 128 + k * L)

        pltpu.sync_copy(cnt_v.at[pl.ds(0, zrows)],
                        cnt_sh.at[pl.ds(s * zrows, zrows)])
        plsc.subcore_barrier()

        def chunk(ch, _):
            base = wid * per_worker + ch * CH
            pltpu.sync_copy(dst_hbm.at[pl.ds(base, CH)], dst_v)
            pltpu.sync_copy(et_hbm.at[pl.ds(base, CH)], et_v)

            def group(g, _):
                d = dst_v[pl.ds(g * L, L)]
                t = et_v[pl.ds(g * L, L)]
                key = d * R + t
                total = jnp.ones((L,), jnp.int32)
                prior = jnp.zeros((L,), jnp.int32)
                for sh in range(1, L):
                    perm = (iota - sh) & (L - 1)
                    rolled = key.at[perm].get(mode="promise_in_bounds")
                    eq = (key == rolled).astype(jnp.int32)
                    total = total + eq
                    prior = prior + jnp.where(iota >= sh, eq, 0)
                plsc.addupdate_scatter(
                    cnt_v, [key >> 7, key & 127], total, mask=prior == 0)
                return 0
            lax.fori_loop(0, CH // L, group, 0)
            return 0
        lax.fori_loop(0, n_chunks, chunk, 0)

        for i in range(CNT_ROWS // 128):
            pltpu.sync_copy(cnt_v.at[pl.ds(i * 128, 128)],
                            cnt_sh.at[ridx_v.at[i]], add=True)
        plsc.subcore_barrier()
        pltpu.sync_copy(cnt_sh.at[pl.ds(s * zrows, zrows)],
                        out_hbm.at[c, pl.ds(s * zrows, zrows)])

    return body


def _edge_scale_kernel(R, EP, CNT_ROWS):
    per_worker = EP // (NC * NS)
    CH = 2560
    n_chunks = per_worker // CH
    mesh = plsc.VectorSubcoreMesh(
        core_axis_name="c", subcore_axis_name="s",
        num_cores=NC, num_subcores=NS)

    @functools.partial(
        pl.kernel,
        out_type=jax.ShapeDtypeStruct((EP,), jnp.float32),
        mesh=mesh,
        compiler_params=pltpu.CompilerParams(needs_layout_passes=False),
        scratch_types=[
            pltpu.VMEM((CNT_ROWS, 128), jnp.float32),
            pltpu.VMEM((CH,), jnp.int32),
            pltpu.VMEM((CH,), jnp.int32),
            pltpu.VMEM((CH,), jnp.float32),
        ],
    )
    def body(dst_hbm, et_hbm, recip_hbm, out_hbm, recip_v, dst_v, et_v, s_v):
        c = lax.axis_index("c")
        s = lax.axis_index("s")
        wid = c * NS + s
        pltpu.sync_copy(recip_hbm, recip_v)

        def chunk(ch, _):
            base = wid * per_worker + ch * CH
            pltpu.sync_copy(dst_hbm.at[pl.ds(base, CH)], dst_v)
            pltpu.sync_copy(et_hbm.at[pl.ds(base, CH)], et_v)

            def group(g, _):
                sl = pl.ds(g * L, L)
                key = dst_v[sl] * R + et_v[sl]
                s_v[sl] = plsc.load_gather(recip_v, [key >> 7, key & 127])
                return 0
            lax.fori_loop(0, CH // L, group, 0)
            pltpu.sync_copy(s_v, out_hbm.at[pl.ds(base, CH)])
            return 0
        lax.fori_loop(0, n_chunks, chunk, 0)

    return body


def _scatter_kernel(R, N, EP, ACC_ROWS):
    RN = R * N
    per_tile = EP // NS
    SUP = 1280
    n_sup = per_tile // SUP
    CHK = 128
    n_chk = SUP // CHK
    arows = ACC_ROWS // NS
    mesh = plsc.VectorSubcoreMesh(
        core_axis_name="c", subcore_axis_name="s",
        num_cores=NC, num_subcores=NS)

    @functools.partial(
        pl.kernel,
        out_type=jax.ShapeDtypeStruct((NC, ACC_ROWS, 128), jnp.float32),
        mesh=mesh,
        compiler_params=pltpu.CompilerParams(needs_layout_passes=False),
        scratch_types=[
            pltpu.VMEM((SUP,), jnp.int32),
            pltpu.VMEM((SUP,), jnp.int32),
            pltpu.VMEM((SUP,), jnp.int32),
            pltpu.VMEM((SUP,), jnp.float32),
            pltpu.VMEM((SUP // 128, 128), jnp.int32),
            pltpu.VMEM((SUP // 128, 128), jnp.int32),
            pltpu.VMEM((128, 128), jnp.float32),
            pltpu.VMEM_SHARED((ACC_ROWS, 128), jnp.float32),
            pltpu.SemaphoreType.DMA,
        ],
    )
    def body(xw_hbm, src_hbm, et_hbm, dst_hbm, s_hbm, out_hbm,
             src_v, et_v, dst_v, sc_v, gidx_v, didx_v, rows_v,
             acc_sh, sem):
        c = lax.axis_index("c")
        s = lax.axis_index("s")
        zero16 = jnp.zeros((L,), jnp.float32)

        def zrow(i, _):
            for k in range(8):
                rows_v[i, pl.ds(k * L, L)] = zero16
            return 0
        lax.fori_loop(0, 128, zrow, 0)
        done = 0
        for j in range((arows + 127) // 128):
            nrows = min(128, arows - done)
            pltpu.sync_copy(rows_v.at[pl.ds(0, nrows)],
                            acc_sh.at[pl.ds(s * arows + done, nrows)])
            done += nrows
        plsc.subcore_barrier()

        lbase = c * RN

        def sup_chunk(sc_i, _):
            base = s * per_tile + sc_i * SUP
            pltpu.sync_copy(src_hbm.at[pl.ds(base, SUP)], src_v)
            pltpu.sync_copy(et_hbm.at[pl.ds(base, SUP)], et_v)
            pltpu.sync_copy(dst_hbm.at[pl.ds(base, SUP)], dst_v)
            pltpu.sync_copy(s_hbm.at[pl.ds(base, SUP)], sc_v)

            def gids(g, _):
                sl = pl.ds(g * L, L)
                t = et_v[sl]
                gidx = lbase + t * N + src_v[sl]
                gidx_v[g >> 3, pl.ds((g & 7) * L, L)] = gidx
                didx_v[g >> 3, pl.ds((g & 7) * L, L)] = dst_v[sl]
                return 0
            lax.fori_loop(0, SUP // L, gids, 0)

            def one_chunk(ch, _):
                pltpu.async_copy(
                    xw_hbm.at[gidx_v.at[ch]], rows_v, sem).wait()

                def edge(e, _):
                    sp = plsc.load_gather(
                        sc_v, [jnp.full((L,), ch * 128 + e, jnp.int32)])
                    for k in range(8):
                        csl = pl.ds(k * L, L)
                        rows_v[e, csl] = rows_v[e, csl] * sp
                    return 0
                lax.fori_loop(0, 128, edge, 0)

                pltpu.sync_copy(rows_v, acc_sh.at[didx_v.at[ch]], add=True)
                return 0
            lax.fori_loop(0, n_chk, one_chunk, 0)
            return 0
        lax.fori_loop(0, n_sup, sup_chunk, 0)

        plsc.subcore_barrier()
        done = 0
        for j in range((arows + 127) // 128):
            nrows = min(128, arows - done)
            pltpu.sync_copy(acc_sh.at[pl.ds(s * arows + done, nrows)],
                            out_hbm.at[c, pl.ds(s * arows + done, nrows)])
            done += nrows

    return body


def kernel(x, edge_index, edge_type, W1, root1, b1, W2, root2, b2):
    N, D = x.shape
    R = W1.shape[0]
    E = edge_index.shape[1]

    UNIT = NC * NS * 2560
    EP = -(-E // UNIT) * UNIT
    pad = EP - E
    src_p = jnp.concatenate([edge_index[0], jnp.zeros((pad,), jnp.int32)])
    et_p = jnp.concatenate([edge_type, jnp.zeros((pad,), jnp.int32)])
    dst_p = jnp.concatenate([edge_index[1], jnp.full((pad,), N, jnp.int32)])

    CNT_ROWS = -(-(N * R + R) // (128 * 128)) * 128
    ACC_ROWS = -(-(N + 1) // (NS * 8)) * (NS * 8)

    cnt_parts = _count_kernel(R, EP, CNT_ROWS)(dst_p, et_p)

    def _recip_body(a_ref, b_ref, o_ref):
        tot = (a_ref[...] + b_ref[...]).astype(jnp.float32)
        o_ref[...] = 1.0 / jnp.maximum(tot, 1.0)
    recip = pl.pallas_call(
        _recip_body,
        out_shape=jax.ShapeDtypeStruct((CNT_ROWS, 128), jnp.float32),
    )(cnt_parts[0], cnt_parts[1])

    Ws = jnp.stack([W1, W2])
    BLK = 1000
    NB = N // BLK

    def _xw_body(x_ref, w_ref, o_ref):
        o_ref[...] = jnp.dot(x_ref[...], w_ref[0, 0],
                             preferred_element_type=jnp.float32)
    xw = pl.pallas_call(
        _xw_body,
        grid=(NB, NC, R),
        in_specs=[
            pl.BlockSpec((BLK, D), lambda i, l, r: (i, 0)),
            pl.BlockSpec((1, 1, D, D), lambda i, l, r: (l, r, 0, 0)),
        ],
        out_specs=pl.BlockSpec(
            (BLK, D), lambda i, l, r: ((l * R + r) * NB + i, 0)),
        out_shape=jax.ShapeDtypeStruct((NC * R * N, D), jnp.float32),
    )(x, Ws)

    s_edge = _edge_scale_kernel(R, EP, CNT_ROWS)(dst_p, et_p, recip)

    msgs = _scatter_kernel(R, N, EP, ACC_ROWS)(
        xw, src_p, et_p, dst_p, s_edge)

    m1 = msgs[0, :N]
    m2 = msgs[1, :N]
    b1r = b1.reshape(1, D)
    b2r = b2.reshape(1, D)

    def _final_body(x_ref, m1_ref, m2_ref, r1_ref, r2_ref, b1_ref, b2_ref,
                    o_ref):
        h = jnp.dot(x_ref[...], r1_ref[...],
                    preferred_element_type=jnp.float32)
        h = jnp.maximum(h + m1_ref[...] + b1_ref[...], 0.0)
        o = jnp.dot(h, r2_ref[...], preferred_element_type=jnp.float32)
        o_ref[...] = o + m2_ref[...] + b2_ref[...]
    out = pl.pallas_call(
        _final_body,
        grid=(NB,),
        in_specs=[
            pl.BlockSpec((BLK, D), lambda i: (i, 0)),
            pl.BlockSpec((BLK, D), lambda i: (i, 0)),
            pl.BlockSpec((BLK, D), lambda i: (i, 0)),
            pl.BlockSpec((D, D), lambda i: (0, 0)),
            pl.BlockSpec((D, D), lambda i: (0, 0)),
            pl.BlockSpec((1, D), lambda i: (0, 0)),
            pl.BlockSpec((1, D), lambda i: (0, 0)),
        ],
        out_specs=pl.BlockSpec((BLK, D), lambda i: (i, 0)),
        out_shape=jax.ShapeDtypeStruct((N, D), jnp.float32),
    )(x, m1, m2, root1, root2, b1r, b2r)
    return out


# speedup vs baseline: 9.7383x; 1.2369x over previous
import functools

import jax
import jax.numpy as jnp
from jax import lax
from jax.experimental import pallas as pl
from jax.experimental.pallas import tpu as pltpu
from jax.experimental.pallas import tpu_sc as plsc

NC, NS, L = 2, 16, 16


def _count_kernel(R, EP, CNT_ROWS):
    per_worker = EP // (NC * NS)
    CH = 2560
    n_chunks = per_worker // CH
    zrows = CNT_ROWS // NS
    mesh = plsc.VectorSubcoreMesh(
        core_axis_name="c", subcore_axis_name="s",
        num_cores=NC, num_subcores=NS)

    @functools.partial(
        pl.kernel,
        out_type=jax.ShapeDtypeStruct((NC, CNT_ROWS, 128), jnp.int32),
        mesh=mesh,
        compiler_params=pltpu.CompilerParams(needs_layout_passes=False),
        scratch_types=[
            pltpu.VMEM((CNT_ROWS, 128), jnp.int32),
            pltpu.VMEM((CH,), jnp.int32),
            pltpu.VMEM((CH,), jnp.int32),
            pltpu.VMEM((CNT_ROWS // 128, 128), jnp.int32),
            pltpu.VMEM_SHARED((CNT_ROWS, 128), jnp.int32),
        ],
    )
    def body(dst_hbm, et_hbm, out_hbm, cnt_v, dst_v, et_v, ridx_v, cnt_sh):
        c = lax.axis_index("c")
        s = lax.axis_index("s")
        wid = c * NS + s
        iota = lax.iota(jnp.int32, L)
        zero16 = jnp.zeros((L,), jnp.int32)

        def zrow(i, _):
            for k in range(8):
                cnt_v[i, pl.ds(k * L, L)] = zero16
            return 0
        lax.fori_loop(0, CNT_ROWS, zrow, 0)
        for i in range(CNT_ROWS // 128):
            for k in range(8):
                ridx_v[i, pl.ds(k * L, L)] = iota + (i * 128 + k * L)

        pltpu.sync_copy(cnt_v.at[pl.ds(0, zrows)],
                        cnt_sh.at[pl.ds(s * zrows, zrows)])
        plsc.subcore_barrier()

        def chunk(ch, _):
            base = wid * per_worker + ch * CH
            pltpu.sync_copy(dst_hbm.at[pl.ds(base, CH)], dst_v)
            pltpu.sync_copy(et_hbm.at[pl.ds(base, CH)], et_v)

            def group(g, _):
                d = dst_v[pl.ds(g * L, L)]
                t = et_v[pl.ds(g * L, L)]
                key = d * R + t
                total = jnp.ones((L,), jnp.int32)
                prior = jnp.zeros((L,), jnp.int32)
                for sh in range(1, L):
                    perm = (iota - sh) & (L - 1)
                    rolled = key.at[perm].get(mode="promise_in_bounds")
                    eq = (key == rolled).astype(jnp.int32)
                    total = total + eq
                    prior = prior + jnp.where(iota >= sh, eq, 0)
                plsc.addupdate_scatter(
                    cnt_v, [key >> 7, key & 127], total, mask=prior == 0)
                return 0
            lax.fori_loop(0, CH // L, group, 0)
            return 0
        lax.fori_loop(0, n_chunks, chunk, 0)

        for i in range(CNT_ROWS // 128):
            pltpu.sync_copy(cnt_v.at[pl.ds(i * 128, 128)],
                            cnt_sh.at[ridx_v.at[i]], add=True)
        plsc.subcore_barrier()
        pltpu.sync_copy(cnt_sh.at[pl.ds(s * zrows, zrows)],
                        out_hbm.at[c, pl.ds(s * zrows, zrows)])

    return body


def _edge_scale_kernel(R, EP, CNT_ROWS):
    per_worker = EP // (NC * NS)
    CH = 2560
    n_chunks = per_worker // CH
    mesh = plsc.VectorSubcoreMesh(
        core_axis_name="c", subcore_axis_name="s",
        num_cores=NC, num_subcores=NS)

    @functools.partial(
        pl.kernel,
        out_type=jax.ShapeDtypeStruct((EP,), jnp.float32),
        mesh=mesh,
        compiler_params=pltpu.CompilerParams(needs_layout_passes=False),
        scratch_types=[
            pltpu.VMEM((CNT_ROWS, 128), jnp.float32),
            pltpu.VMEM((CH,), jnp.int32),
            pltpu.VMEM((CH,), jnp.int32),
            pltpu.VMEM((CH,), jnp.float32),
        ],
    )
    def body(dst_hbm, et_hbm, recip_hbm, out_hbm, recip_v, dst_v, et_v, s_v):
        c = lax.axis_index("c")
        s = lax.axis_index("s")
        wid = c * NS + s
        pltpu.sync_copy(recip_hbm, recip_v)

        def chunk(ch, _):
            base = wid * per_worker + ch * CH
            pltpu.sync_copy(dst_hbm.at[pl.ds(base, CH)], dst_v)
            pltpu.sync_copy(et_hbm.at[pl.ds(base, CH)], et_v)

            def group(g, _):
                sl = pl.ds(g * L, L)
                key = dst_v[sl] * R + et_v[sl]
                s_v[sl] = plsc.load_gather(recip_v, [key >> 7, key & 127])
                return 0
            lax.fori_loop(0, CH // L, group, 0)
            pltpu.sync_copy(s_v, out_hbm.at[pl.ds(base, CH)])
            return 0
        lax.fori_loop(0, n_chunks, chunk, 0)

    return body


def _scatter_kernel(R, N, EP, ACC_ROWS):
    RN = R * N
    per_tile = EP // NS
    SUP = 1280
    n_sup = per_tile // SUP
    CHK = 128
    n_chk = SUP // CHK
    arows = ACC_ROWS // NS
    mesh = plsc.VectorSubcoreMesh(
        core_axis_name="c", subcore_axis_name="s",
        num_cores=NC, num_subcores=NS)

    @functools.partial(
        pl.kernel,
        out_type=jax.ShapeDtypeStruct((NC, ACC_ROWS, 128), jnp.float32),
        mesh=mesh,
        compiler_params=pltpu.CompilerParams(needs_layout_passes=False),
        scratch_types=[
            pltpu.VMEM((SUP,), jnp.int32),
            pltpu.VMEM((SUP,), jnp.int32),
            pltpu.VMEM((SUP,), jnp.int32),
            pltpu.VMEM((SUP,), jnp.float32),
            pltpu.VMEM((SUP // 128, 128), jnp.int32),
            pltpu.VMEM((SUP // 128, 128), jnp.int32),
            pltpu.VMEM((2 * 128, 128), jnp.float32),
            pltpu.VMEM_SHARED((ACC_ROWS, 128), jnp.float32),
            pltpu.SemaphoreType.DMA,
            pltpu.SemaphoreType.DMA,
            pltpu.SemaphoreType.DMA,
        ],
    )
    def body(xw_hbm, src_hbm, et_hbm, dst_hbm, s_hbm, out_hbm,
             src_v, et_v, dst_v, sc_v, gidx_v, didx_v, rows_v,
             acc_sh, sem, gsem0, gsem1):
        c = lax.axis_index("c")
        s = lax.axis_index("s")
        zero16 = jnp.zeros((L,), jnp.float32)

        def zrow(i, _):
            for k in range(8):
                rows_v[i, pl.ds(k * L, L)] = zero16
            return 0
        lax.fori_loop(0, 128, zrow, 0)
        done = 0
        for j in range((arows + 127) // 128):
            nrows = min(128, arows - done)
            pltpu.sync_copy(rows_v.at[pl.ds(0, nrows)],
                            acc_sh.at[pl.ds(s * arows + done, nrows)])
            done += nrows
        plsc.subcore_barrier()

        lbase = c * RN

        def sup_chunk(sc_i, _):
            base = s * per_tile + sc_i * SUP
            pltpu.sync_copy(src_hbm.at[pl.ds(base, SUP)], src_v)
            pltpu.sync_copy(et_hbm.at[pl.ds(base, SUP)], et_v)
            pltpu.sync_copy(dst_hbm.at[pl.ds(base, SUP)], dst_v)
            pltpu.sync_copy(s_hbm.at[pl.ds(base, SUP)], sc_v)

            def gids(g, _):
                sl = pl.ds(g * L, L)
                t = et_v[sl]
                gidx = lbase + t * N + src_v[sl]
                gidx_v[g >> 3, pl.ds((g & 7) * L, L)] = gidx
                didx_v[g >> 3, pl.ds((g & 7) * L, L)] = dst_v[sl]
                return 0
            lax.fori_loop(0, SUP // L, gids, 0)

            pltpu.async_copy(
                xw_hbm.at[gidx_v.at[0]], rows_v.at[pl.ds(0, 128)], gsem0)
            for ch in range(n_chk):
                b = ch % 2
                bsl = pl.ds(b * 128, 128)
                gsem = gsem0 if b == 0 else gsem1
                pltpu.make_async_copy(
                    xw_hbm.at[gidx_v.at[ch]], rows_v.at[bsl], gsem).wait()
                if ch + 1 < n_chk:
                    ob = 1 - b
                    obsl = pl.ds(ob * 128, 128)
                    osem = gsem0 if ob == 0 else gsem1
                    pltpu.async_copy(
                        xw_hbm.at[gidx_v.at[ch + 1]], rows_v.at[obsl], osem)

                def edge(e, _, _ch=ch, _b=b):
                    sp = plsc.load_gather(
                        sc_v, [jnp.full((L,), _ch * 128 + e, jnp.int32)])
                    row = _b * 128 + e
                    for k in range(8):
                        csl = pl.ds(k * L, L)
                        rows_v[row, csl] = rows_v[row, csl] * sp
                    return 0
                lax.fori_loop(0, 128, edge, 0, unroll=2)

                pltpu.sync_copy(rows_v.at[bsl],
                                acc_sh.at[didx_v.at[ch]], add=True)
            return 0
        lax.fori_loop(0, n_sup, sup_chunk, 0)

        plsc.subcore_barrier()
        done = 0
        for j in range((arows + 127) // 128):
            nrows = min(128, arows - done)
            pltpu.sync_copy(acc_sh.at[pl.ds(s * arows + done, nrows)],
                            out_hbm.at[c, pl.ds(s * arows + done, nrows)])
            done += nrows

    return body


def kernel(x, edge_index, edge_type, W1, root1, b1, W2, root2, b2):
    N, D = x.shape
    R = W1.shape[0]
    E = edge_index.shape[1]

    UNIT = NC * NS * 2560
    EP = -(-E // UNIT) * UNIT
    pad = EP - E
    src_p = jnp.concatenate([edge_index[0], jnp.zeros((pad,), jnp.int32)])
    et_p = jnp.concatenate([edge_type, jnp.zeros((pad,), jnp.int32)])
    dst_p = jnp.concatenate([edge_index[1], jnp.full((pad,), N, jnp.int32)])

    CNT_ROWS = -(-(N * R + R) // (128 * 128)) * 128
    ACC_ROWS = -(-(N + 1) // (NS * 8)) * (NS * 8)

    cnt_parts = _count_kernel(R, EP, CNT_ROWS)(dst_p, et_p)

    def _recip_body(a_ref, b_ref, o_ref):
        tot = (a_ref[...] + b_ref[...]).astype(jnp.float32)
        o_ref[...] = 1.0 / jnp.maximum(tot, 1.0)
    recip = pl.pallas_call(
        _recip_body,
        out_shape=jax.ShapeDtypeStruct((CNT_ROWS, 128), jnp.float32),
    )(cnt_parts[0], cnt_parts[1])

    Ws = jnp.stack([W1, W2])
    BLK = 1000
    NB = N // BLK

    def _xw_body(x_ref, w_ref, o_ref):
        o_ref[...] = jnp.dot(x_ref[...], w_ref[0, 0],
                             preferred_element_type=jnp.float32)
    xw = pl.pallas_call(
        _xw_body,
        grid=(NB, NC, R),
        in_specs=[
            pl.BlockSpec((BLK, D), lambda i, l, r: (i, 0)),
            pl.BlockSpec((1, 1, D, D), lambda i, l, r: (l, r, 0, 0)),
        ],
        out_specs=pl.BlockSpec(
            (BLK, D), lambda i, l, r: ((l * R + r) * NB + i, 0)),
        out_shape=jax.ShapeDtypeStruct((NC * R * N, D), jnp.float32),
    )(x, Ws)

    s_edge = _edge_scale_kernel(R, EP, CNT_ROWS)(dst_p, et_p, recip)

    msgs = _scatter_kernel(R, N, EP, ACC_ROWS)(
        xw, src_p, et_p, dst_p, s_edge)

    m1 = msgs[0, :N]
    m2 = msgs[1, :N]
    b1r = b1.reshape(1, D)
    b2r = b2.reshape(1, D)

    def _final_body(x_ref, m1_ref, m2_ref, r1_ref, r2_ref, b1_ref, b2_ref,
                    o_ref):
        h = jnp.dot(x_ref[...], r1_ref[...],
                    preferred_element_type=jnp.float32)
        h = jnp.maximum(h + m1_ref[...] + b1_ref[...], 0.0)
        o = jnp.dot(h, r2_ref[...], preferred_element_type=jnp.float32)
        o_ref[...] = o + m2_ref[...] + b2_ref[...]
    out = pl.pallas_call(
        _final_body,
        grid=(NB,),
        in_specs=[
            pl.BlockSpec((BLK, D), lambda i: (i, 0)),
            pl.BlockSpec((BLK, D), lambda i: (i, 0)),
            pl.BlockSpec((BLK, D), lambda i: (i, 0)),
            pl.BlockSpec((D, D), lambda i: (0, 0)),
            pl.BlockSpec((D, D), lambda i: (0, 0)),
            pl.BlockSpec((1, D), lambda i: (0, 0)),
            pl.BlockSpec((1, D), lambda i: (0, 0)),
        ],
        out_specs=pl.BlockSpec((BLK, D), lambda i: (i, 0)),
        out_shape=jax.ShapeDtypeStruct((N, D), jnp.float32),
    )(x, m1, m2, root1, root2, b1r, b2r)
    return out
